# Initial kernel scaffold; baseline (speedup 1.0000x reference)
import jax
import jax.numpy as jnp
from jax.experimental import pallas as pl


def kernel(x0, x1, x2, edge_index0, edge_index1, edge_index2, batch0, batch1, batch2, W, b):
    raise NotImplementedError("write your pallas kernel here")


# speedup vs baseline: 7.6301x; 7.6301x over previous
import functools

import jax
import jax.numpy as jnp
from jax import lax
from jax.experimental import pallas as pl
from jax.experimental.pallas import tpu as pltpu
from jax.experimental.pallas import tpu_sc as plsc

NUM_LAYERS = 3
LANES = 16
NTILES = 16
NCORES = 2
CHUNK = 128
BATCH = 64


def _round_up(a: int, m: int) -> int:
    return (a + m - 1) // m * m


def _mm_body(x_ref, w_ref, o_ref):
    o_ref[...] = jnp.dot(x_ref[...], w_ref[...],
                         preferred_element_type=jnp.float32)


def _project(xpad, wp):
    npad, d = xpad.shape
    bm = 512
    return pl.pallas_call(
        _mm_body,
        grid=(npad // bm,),
        in_specs=[
            pl.BlockSpec((bm, d), lambda i: (i, 0)),
            pl.BlockSpec((d, LANES), lambda i: (0, 0)),
        ],
        out_specs=pl.BlockSpec((bm, LANES), lambda i: (i, 0)),
        out_shape=jax.ShapeDtypeStruct((npad, LANES), jnp.float32),
    )(xpad, wp)


def _make_sc_kernel(npads, epads):
    rpts = [n // NTILES for n in npads]
    max_rpt = max(rpts)
    mesh = plsc.VectorSubcoreMesh(core_axis_name="c", subcore_axis_name="s")

    @functools.partial(
        pl.kernel,
        out_type=jax.ShapeDtypeStruct((NCORES, BATCH, LANES), jnp.float32),
        mesh=mesh,
        compiler_params=pltpu.CompilerParams(use_tc_tiling_on_sc=False),
        scratch_types=dict(
            zA0=pltpu.VMEM_SHARED((npads[0], LANES), jnp.float32),
            zB0=pltpu.VMEM_SHARED((npads[0], LANES), jnp.float32),
            zA1=pltpu.VMEM_SHARED((npads[1], LANES), jnp.float32),
            zB1=pltpu.VMEM_SHARED((npads[1], LANES), jnp.float32),
            zA2=pltpu.VMEM_SHARED((npads[2], LANES), jnp.float32),
            zB2=pltpu.VMEM_SHARED((npads[2], LANES), jnp.float32),
            pooled=pltpu.VMEM_SHARED((BATCH, LANES), jnp.float32),
            sbuf=pltpu.VMEM((CHUNK,), jnp.int32),
            dbuf=pltpu.VMEM((CHUNK,), jnp.int32),
            rbuf=pltpu.VMEM((CHUNK, LANES), jnp.float32),
            stage=pltpu.VMEM((max_rpt, LANES), jnp.float32),
            pstage=pltpu.VMEM((BATCH, LANES), jnp.float32),
            sem=pltpu.SemaphoreType.DMA,
        ),
    )
    def sc_kernel(z0, z1, z2, s0, d0, s1, d1, s2, d2, b0, b1, b2, out,
                  zA0, zB0, zA1, zB1, zA2, zB2, pooled,
                  sbuf, dbuf, rbuf, stage, pstage, sem):
        c = lax.axis_index("c")
        s = lax.axis_index("s")

        def load_dim(z_hbm, zA, zB, rpt):
            r0 = s * rpt
            pltpu.sync_copy(z_hbm.at[pl.ds(r0, rpt)], stage.at[pl.ds(0, rpt)])
            pltpu.sync_copy(stage.at[pl.ds(0, rpt)], zA.at[pl.ds(r0, rpt)])
            pltpu.sync_copy(stage.at[pl.ds(0, rpt)], zB.at[pl.ds(r0, rpt)])

        def copy_dim(zsrc, zdst, rpt):
            r0 = s * rpt
            pltpu.sync_copy(zsrc.at[pl.ds(r0, rpt)], stage.at[pl.ds(0, rpt)])
            pltpu.sync_copy(stage.at[pl.ds(0, rpt)], zdst.at[pl.ds(r0, rpt)])

        def edge_pass(zsrc, zdst, src_hbm, dst_hbm, epad):
            ept = epad // NTILES
            base = s * ept

            def it(j, carry):
                off = base + j * CHUNK
                pltpu.sync_copy(src_hbm.at[pl.ds(off, CHUNK)], sbuf)
                pltpu.sync_copy(dst_hbm.at[pl.ds(off, CHUNK)], dbuf)
                pltpu.async_copy(zsrc.at[sbuf], rbuf, sem).wait()
                pltpu.sync_copy(rbuf, zdst.at[dbuf], add=True)
                return carry

            lax.fori_loop(0, ept // CHUNK, it, 0)

        def pool_dim(zfin, b_hbm, rpt):
            base = s * rpt

            def it(j, carry):
                off = base + j * CHUNK
                pltpu.sync_copy(b_hbm.at[pl.ds(off, CHUNK)], dbuf)
                pltpu.sync_copy(zfin.at[pl.ds(off, CHUNK)], rbuf)
                pltpu.sync_copy(rbuf, pooled.at[dbuf], add=True)
                return carry

            lax.fori_loop(0, rpt // CHUNK, it, 0)

        def run_dims(dims):
            for (zh, zA, zB, sh, dh, bh, epad, rpt) in dims:
                load_dim(zh, zA, zB, rpt)

            @pl.when(s == 0)
            def _():
                zv = jnp.zeros((LANES,), jnp.float32)
                for i in range(BATCH):
                    pstage[i, :] = zv
                pltpu.sync_copy(pstage, pooled)

            plsc.subcore_barrier()

            for layer in range(NUM_LAYERS):
                fwd = layer % 2 == 0
                if layer > 0:
                    for (zh, zA, zB, sh, dh, bh, epad, rpt) in dims:
                        copy_dim(zA if fwd else zB, zB if fwd else zA, rpt)
                    plsc.subcore_barrier()
                for (zh, zA, zB, sh, dh, bh, epad, rpt) in dims:
                    edge_pass(zA if fwd else zB, zB if fwd else zA, sh, dh, epad)
                plsc.subcore_barrier()

            for (zh, zA, zB, sh, dh, bh, epad, rpt) in dims:
                pool_dim(zB if NUM_LAYERS % 2 == 1 else zA, bh, rpt)
            plsc.subcore_barrier()

            @pl.when(s == 0)
            def _():
                pltpu.sync_copy(pooled, pstage)
                pltpu.sync_copy(pstage, out.at[c])

        dim0 = (z0, zA0, zB0, s0, d0, b0, epads[0], rpts[0])
        dim1 = (z1, zA1, zB1, s1, d1, b1, epads[1], rpts[1])
        dim2 = (z2, zA2, zB2, s2, d2, b2, epads[2], rpts[2])

        @pl.when(c == 0)
        def _():
            run_dims([dim0])

        @pl.when(c == 1)
        def _():
            run_dims([dim1, dim2])

    return sc_kernel


def kernel(x0, x1, x2, edge_index0, edge_index1, edge_index2,
           batch0, batch1, batch2, W, b):
    xs = [x0, x1, x2]
    eis = [edge_index0, edge_index1, edge_index2]
    bs = [batch0, batch1, batch2]
    ns = [x.shape[0] for x in xs]
    npads = [_round_up(n + 1, NTILES * CHUNK) for n in ns]
    epads = [_round_up(ei.shape[1], NTILES * CHUNK) for ei in eis]

    wp = jnp.pad(W, ((0, 0), (0, LANES - W.shape[1])))
    zs = [_project(jnp.pad(x, ((0, npads[i] - ns[i]), (0, 0))), wp)
          for i, x in enumerate(xs)]
    srcs = [jnp.concatenate([eis[i][0], jnp.full((epads[i] - eis[i].shape[1],),
                                                 ns[i], jnp.int32)])
            for i in range(3)]
    dsts = [jnp.concatenate([eis[i][1], jnp.full((epads[i] - eis[i].shape[1],),
                                                 ns[i], jnp.int32)])
            for i in range(3)]
    bpads = [jnp.pad(bs[i], (0, npads[i] - ns[i])) for i in range(3)]

    sc = _make_sc_kernel(npads, epads)
    pooled2 = sc(zs[0], zs[1], zs[2],
                 srcs[0], dsts[0], srcs[1], dsts[1], srcs[2], dsts[2],
                 bpads[0], bpads[1], bpads[2])
    return pooled2.sum(axis=0)[:, : W.shape[1]] + b


# speedup vs baseline: 21.1796x; 2.7758x over previous
import functools

import jax
import jax.numpy as jnp
from jax import lax
from jax.experimental import pallas as pl
from jax.experimental.pallas import tpu as pltpu
from jax.experimental.pallas import tpu_sc as plsc

NUM_LAYERS = 3
LANES = 16
NTILES = 16
NCORES = 2
CHUNK = 128
BATCH = 64


def _round_up(a: int, m: int) -> int:
    return (a + m - 1) // m * m


def _mm_body(x_ref, w_ref, o_ref):
    o_ref[...] = jnp.dot(x_ref[...], w_ref[...],
                         preferred_element_type=jnp.float32)


def _project(xpad, wp):
    npad, d = xpad.shape
    bm = 512
    return pl.pallas_call(
        _mm_body,
        grid=(npad // bm,),
        in_specs=[
            pl.BlockSpec((bm, d), lambda i: (i, 0)),
            pl.BlockSpec((d, LANES), lambda i: (0, 0)),
        ],
        out_specs=pl.BlockSpec((bm, LANES), lambda i: (i, 0)),
        out_shape=jax.ShapeDtypeStruct((npad, LANES), jnp.float32),
    )(xpad, wp)


NBUF = 4


def _make_sc_kernel(npads, epads):
    rpts = [n // NTILES for n in npads]
    max_rpt = max(rpts)
    ncts = [e // NTILES // CHUNK for e in epads]
    mesh = plsc.VectorSubcoreMesh(core_axis_name="c", subcore_axis_name="s")

    scr = dict(
        zA0=pltpu.VMEM_SHARED((npads[0], LANES), jnp.float32),
        zB0=pltpu.VMEM_SHARED((npads[0], LANES), jnp.float32),
        zA1=pltpu.VMEM_SHARED((npads[1], LANES), jnp.float32),
        zB1=pltpu.VMEM_SHARED((npads[1], LANES), jnp.float32),
        zA2=pltpu.VMEM_SHARED((npads[2], LANES), jnp.float32),
        zB2=pltpu.VMEM_SHARED((npads[2], LANES), jnp.float32),
        pooled=pltpu.VMEM_SHARED((BATCH, LANES), jnp.float32),
        eir0=pltpu.VMEM((NBUF, 2, CHUNK), jnp.int32),
        eir1=pltpu.VMEM((NBUF, 2, CHUNK), jnp.int32),
        dbuf=pltpu.VMEM((CHUNK,), jnp.int32),
        stage=pltpu.VMEM((CHUNK, LANES), jnp.float32),
        pstage=pltpu.VMEM((BATCH, LANES), jnp.float32),
        isem0=pltpu.SemaphoreType.DMA,
        isem1=pltpu.SemaphoreType.DMA,
    )
    for bi in range(NBUF):
        scr[f"rbuf{bi}"] = pltpu.VMEM((CHUNK, LANES), jnp.float32)
    for bi in range(NBUF):
        scr[f"gsem{bi}"] = pltpu.SemaphoreType.DMA
    for bi in range(NBUF):
        scr[f"ssem{bi}"] = pltpu.SemaphoreType.DMA

    @functools.partial(
        pl.kernel,
        out_type=jax.ShapeDtypeStruct((NCORES, BATCH, LANES), jnp.float32),
        mesh=mesh,
        compiler_params=pltpu.CompilerParams(use_tc_tiling_on_sc=False),
        scratch_types=scr,
    )
    def sc_kernel(z0, z1, z2, e0, e1, e2, b0, b1, b2, out,
                  zA0, zB0, zA1, zB1, zA2, zB2, pooled,
                  eir0, eir1, dbuf, stage, pstage, isem0, isem1,
                  rbuf0, rbuf1, rbuf2, rbuf3,
                  gsem0, gsem1, gsem2, gsem3,
                  ssem0, ssem1, ssem2, ssem3):
        c = lax.axis_index("c")
        s = lax.axis_index("s")
        rbufs = [rbuf0, rbuf1, rbuf2, rbuf3]
        gsems = [gsem0, gsem1, gsem2, gsem3]
        ssems = [ssem0, ssem1, ssem2, ssem3]

        def rows_2hop(src, dst, rpt):
            base = s * rpt

            def it(j, carry):
                r0 = base + j * CHUNK
                pltpu.sync_copy(src.at[pl.ds(r0, CHUNK)], stage)
                pltpu.sync_copy(stage, dst.at[pl.ds(r0, CHUNK)])
                return carry

            lax.fori_loop(0, rpt // CHUNK, it, 0)

        def load_dim(z_hbm, zA, zB, rpt):
            base = s * rpt

            def it(j, carry):
                r0 = base + j * CHUNK
                pltpu.sync_copy(z_hbm.at[pl.ds(r0, CHUNK)], stage)
                pltpu.sync_copy(stage, zA.at[pl.ds(r0, CHUNK)])
                pltpu.sync_copy(stage, zB.at[pl.ds(r0, CHUNK)])
                return carry

            lax.fori_loop(0, rpt // CHUNK, it, 0)

        def process_quad(zsrc, zdst, eir):
            gds = [
                pltpu.async_copy(zsrc.at[eir.at[bi, 0]], rbufs[bi], gsems[bi])
                for bi in range(NBUF)
            ]
            sds = []
            for bi in range(NBUF):
                gds[bi].wait()
                sds.append(pltpu.async_copy(
                    rbufs[bi], zdst.at[eir.at[bi, 1]], ssems[bi], add=True))
            for sd in sds:
                sd.wait()

        def edge_pass(zsrc, zdst, e_hbm, nct):
            base = s * nct
            pltpu.sync_copy(e_hbm.at[pl.ds(base, NBUF)], eir0)

            def pair(h, carry):
                q1 = base + (2 * h + 1) * NBUF
                dB = pltpu.async_copy(e_hbm.at[pl.ds(q1, NBUF)], eir1, isem1)
                process_quad(zsrc, zdst, eir0)
                dB.wait()
                dA = pltpu.async_copy(e_hbm.at[pl.ds(q1 + NBUF, NBUF)],
                                      eir0, isem0)
                process_quad(zsrc, zdst, eir1)
                dA.wait()
                return carry

            lax.fori_loop(0, nct // NBUF // 2, pair, 0)

        def pool_dim(zfin, b_hbm, rpt):
            base = s * rpt

            def it(j, carry):
                off = base + j * CHUNK
                pltpu.sync_copy(b_hbm.at[pl.ds(off, CHUNK)], dbuf)
                pltpu.sync_copy(zfin.at[pl.ds(off, CHUNK)], rbufs[0])
                pltpu.sync_copy(rbufs[0], pooled.at[dbuf], add=True)
                return carry

            lax.fori_loop(0, rpt // CHUNK, it, 0)

        def run_dims(dims):
            for (zh, zA, zB, eh, bh, nct, rpt) in dims:
                load_dim(zh, zA, zB, rpt)

            @pl.when(s == 0)
            def _():
                zv = jnp.zeros((LANES,), jnp.float32)
                for i in range(BATCH):
                    pstage[i, :] = zv
                pltpu.sync_copy(pstage, pooled)

            plsc.subcore_barrier()

            for layer in range(NUM_LAYERS):
                fwd = layer % 2 == 0
                if layer > 0:
                    for (zh, zA, zB, eh, bh, nct, rpt) in dims:
                        rows_2hop(zA if fwd else zB, zB if fwd else zA, rpt)
                    plsc.subcore_barrier()
                for (zh, zA, zB, eh, bh, nct, rpt) in dims:
                    edge_pass(zA if fwd else zB, zB if fwd else zA, eh, nct)
                plsc.subcore_barrier()

            for (zh, zA, zB, eh, bh, nct, rpt) in dims:
                pool_dim(zB if NUM_LAYERS % 2 == 1 else zA, bh, rpt)
            plsc.subcore_barrier()

            @pl.when(s == 0)
            def _():
                pltpu.sync_copy(pooled, pstage)
                pltpu.sync_copy(pstage, out.at[c])

        dim0 = (z0, zA0, zB0, e0, b0, ncts[0], rpts[0])
        dim1 = (z1, zA1, zB1, e1, b1, ncts[1], rpts[1])
        dim2 = (z2, zA2, zB2, e2, b2, ncts[2], rpts[2])

        @pl.when(c == 0)
        def _():
            run_dims([dim0])

        @pl.when(c == 1)
        def _():
            run_dims([dim1, dim2])

    return sc_kernel


def kernel(x0, x1, x2, edge_index0, edge_index1, edge_index2,
           batch0, batch1, batch2, W, b):
    xs = [x0, x1, x2]
    eis = [edge_index0, edge_index1, edge_index2]
    bs = [batch0, batch1, batch2]
    ns = [x.shape[0] for x in xs]
    npads = [_round_up(n + 1, NTILES * CHUNK) for n in ns]
    epads = [_round_up(ei.shape[1], NTILES * CHUNK * NBUF * 2) for ei in eis]

    wp = jnp.pad(W, ((0, 0), (0, LANES - W.shape[1])))
    zs = [_project(jnp.pad(x, ((0, npads[i] - ns[i]), (0, 0))), wp)
          for i, x in enumerate(xs)]
    es = []
    for i in range(3):
        alloc = epads[i] + NBUF * CHUNK
        pad = alloc - eis[i].shape[1]
        ep = jnp.pad(eis[i], ((0, 0), (0, pad)), constant_values=ns[i])
        es.append(ep.reshape(2, alloc // CHUNK, CHUNK).transpose(1, 0, 2))
    bpads = [jnp.pad(bs[i], (0, npads[i] - ns[i])) for i in range(3)]

    sc = _make_sc_kernel(npads, epads)
    pooled2 = sc(zs[0], zs[1], zs[2], es[0], es[1], es[2],
                 bpads[0], bpads[1], bpads[2])
    return pooled2.sum(axis=0)[:, : W.shape[1]] + b


# speedup vs baseline: 23.2481x; 1.0977x over previous
import functools

import jax
import jax.numpy as jnp
from jax import lax
from jax.experimental import pallas as pl
from jax.experimental.pallas import tpu as pltpu
from jax.experimental.pallas import tpu_sc as plsc

NUM_LAYERS = 3
LANES = 16
NTILES = 16
NCORES = 2
CHUNK = 128
NBUF = 8
BATCH = 64


def _round_up(a: int, m: int) -> int:
    return (a + m - 1) // m * m


def _mm_body(x_ref, w_ref, o_ref):
    o_ref[...] = jnp.dot(x_ref[...], w_ref[...],
                         preferred_element_type=jnp.float32)


def _project(xpad, wp):
    npad, d = xpad.shape
    bm = 512
    return pl.pallas_call(
        _mm_body,
        grid=(npad // bm,),
        in_specs=[
            pl.BlockSpec((bm, d), lambda i: (i, 0)),
            pl.BlockSpec((d, LANES), lambda i: (0, 0)),
        ],
        out_specs=pl.BlockSpec((bm, LANES), lambda i: (i, 0)),
        out_shape=jax.ShapeDtypeStruct((npad, LANES), jnp.float32),
    )(xpad, wp)


def _make_sc_kernel(npads, epads):
    rpts = [n // NTILES for n in npads]
    ncts = [e // NTILES // CHUNK for e in epads]
    mesh = plsc.VectorSubcoreMesh(core_axis_name="c", subcore_axis_name="s")

    scr = dict(
        zA0=pltpu.VMEM_SHARED((npads[0], LANES), jnp.float32),
        zB0=pltpu.VMEM_SHARED((npads[0], LANES), jnp.float32),
        zA1=pltpu.VMEM_SHARED((npads[1], LANES), jnp.float32),
        zB1=pltpu.VMEM_SHARED((npads[1], LANES), jnp.float32),
        zA2=pltpu.VMEM_SHARED((npads[2], LANES), jnp.float32),
        zB2=pltpu.VMEM_SHARED((npads[2], LANES), jnp.float32),
        pooled=pltpu.VMEM_SHARED((BATCH, LANES), jnp.float32),
        eir0=pltpu.VMEM((NBUF, 2, CHUNK), jnp.int32),
        eir1=pltpu.VMEM((NBUF, 2, CHUNK), jnp.int32),
        pstage=pltpu.VMEM((BATCH, LANES), jnp.float32),
        isem0=pltpu.SemaphoreType.DMA,
        isem1=pltpu.SemaphoreType.DMA,
    )
    for bi in range(NBUF):
        scr[f"rbuf{bi}"] = pltpu.VMEM((CHUNK, LANES), jnp.float32)
    for bi in range(NBUF):
        scr[f"gsem{bi}"] = pltpu.SemaphoreType.DMA
    for bi in range(NBUF):
        scr[f"ssem{bi}"] = pltpu.SemaphoreType.DMA

    @functools.partial(
        pl.kernel,
        out_type=jax.ShapeDtypeStruct((NCORES, BATCH, LANES), jnp.float32),
        mesh=mesh,
        compiler_params=pltpu.CompilerParams(use_tc_tiling_on_sc=False),
        scratch_types=scr,
    )
    def sc_kernel(z0, z1, z2, e0, e1, e2, b0, b1, b2, out,
                  zA0, zB0, zA1, zB1, zA2, zB2, pooled,
                  eir0, eir1, pstage, isem0, isem1,
                  rbuf0, rbuf1, rbuf2, rbuf3, rbuf4, rbuf5, rbuf6, rbuf7,
                  gsem0, gsem1, gsem2, gsem3, gsem4, gsem5, gsem6, gsem7,
                  ssem0, ssem1, ssem2, ssem3, ssem4, ssem5, ssem6, ssem7):
        c = lax.axis_index("c")
        s = lax.axis_index("s")
        rbufs = [rbuf0, rbuf1, rbuf2, rbuf3, rbuf4, rbuf5, rbuf6, rbuf7]
        gsems = [gsem0, gsem1, gsem2, gsem3, gsem4, gsem5, gsem6, gsem7]
        ssems = [ssem0, ssem1, ssem2, ssem3, ssem4, ssem5, ssem6, ssem7]

        def grouped(n, issue_load, after_load):
            for g0 in range(0, n, NBUF):
                g = min(NBUF, n - g0)
                lds = [issue_load(g0 + i, i) for i in range(g)]
                sds = []
                for i in range(g):
                    lds[i].wait()
                    sds.extend(after_load(g0 + i, i))
                for sd in sds:
                    sd.wait()

        def load_dim(z_hbm, zA, zB, rpt):
            base = s * rpt

            def ld(j, i):
                sl = pl.ds(base + j * CHUNK, CHUNK)
                return pltpu.async_copy(z_hbm.at[sl], rbufs[i], gsems[i])

            def st(j, i):
                sl = pl.ds(base + j * CHUNK, CHUNK)
                return [pltpu.async_copy(rbufs[i], zA.at[sl], ssems[i]),
                        pltpu.async_copy(rbufs[i], zB.at[sl], gsems[i])]

            grouped(rpt // CHUNK, ld, st)

        def copy_rows(src, dst, rpt):
            base = s * rpt

            def ld(j, i):
                sl = pl.ds(base + j * CHUNK, CHUNK)
                return pltpu.async_copy(src.at[sl], rbufs[i], gsems[i])

            def st(j, i):
                sl = pl.ds(base + j * CHUNK, CHUNK)
                return [pltpu.async_copy(rbufs[i], dst.at[sl], ssems[i])]

            grouped(rpt // CHUNK, ld, st)

        def process_quad(zsrc, zdst, eir):
            gds = [
                pltpu.async_copy(zsrc.at[eir.at[bi, 0]], rbufs[bi], gsems[bi])
                for bi in range(NBUF)
            ]
            sds = []
            for bi in range(NBUF):
                gds[bi].wait()
                sds.append(pltpu.async_copy(
                    rbufs[bi], zdst.at[eir.at[bi, 1]], ssems[bi], add=True))
            for sd in sds:
                sd.wait()

        def edge_pass(zsrc, zdst, e_hbm, nct):
            base = s * nct
            nq = nct // NBUF
            pltpu.sync_copy(e_hbm.at[pl.ds(base, NBUF)], eir0)

            def pair(h, carry):
                q1 = base + (2 * h + 1) * NBUF
                dB = pltpu.async_copy(e_hbm.at[pl.ds(q1, NBUF)], eir1, isem1)
                process_quad(zsrc, zdst, eir0)
                dB.wait()
                dA = pltpu.async_copy(e_hbm.at[pl.ds(q1 + NBUF, NBUF)],
                                      eir0, isem0)
                process_quad(zsrc, zdst, eir1)
                dA.wait()
                return carry

            lax.fori_loop(0, nq // 2, pair, 0)
            if nq % 2 == 1:
                process_quad(zsrc, zdst, eir0)

        def pool_dim(zfin, b_hbm, rpt):
            base = s * rpt

            def ld(j, i):
                sl = pl.ds(base + j * CHUNK, CHUNK)
                pltpu.sync_copy(b_hbm.at[sl], eir0.at[i, 0])
                return pltpu.async_copy(zfin.at[sl], rbufs[i], gsems[i])

            def st(j, i):
                return [pltpu.async_copy(rbufs[i], pooled.at[eir0.at[i, 0]],
                                         ssems[i], add=True)]

            grouped(rpt // CHUNK, ld, st)

        def run_dims(dims):
            for (zh, zA, zB, eh, bh, nct, rpt) in dims:
                load_dim(zh, zA, zB, rpt)

            @pl.when(s == 0)
            def _():
                zv = jnp.zeros((LANES,), jnp.float32)
                for i in range(BATCH):
                    pstage[i, :] = zv
                pltpu.sync_copy(pstage, pooled)

            plsc.subcore_barrier()

            for layer in range(NUM_LAYERS):
                fwd = layer % 2 == 0
                if layer > 0:
                    for (zh, zA, zB, eh, bh, nct, rpt) in dims:
                        copy_rows(zA if fwd else zB, zB if fwd else zA, rpt)
                    plsc.subcore_barrier()
                for (zh, zA, zB, eh, bh, nct, rpt) in dims:
                    edge_pass(zA if fwd else zB, zB if fwd else zA, eh, nct)
                plsc.subcore_barrier()

            for (zh, zA, zB, eh, bh, nct, rpt) in dims:
                pool_dim(zB if NUM_LAYERS % 2 == 1 else zA, bh, rpt)
            plsc.subcore_barrier()

            @pl.when(s == 0)
            def _():
                pltpu.sync_copy(pooled, pstage)
                pltpu.sync_copy(pstage, out.at[c])

        dim0 = (z0, zA0, zB0, e0, b0, ncts[0], rpts[0])
        dim1 = (z1, zA1, zB1, e1, b1, ncts[1], rpts[1])
        dim2 = (z2, zA2, zB2, e2, b2, ncts[2], rpts[2])

        @pl.when(c == 0)
        def _():
            run_dims([dim0])

        @pl.when(c == 1)
        def _():
            run_dims([dim1, dim2])

    return sc_kernel


def kernel(x0, x1, x2, edge_index0, edge_index1, edge_index2,
           batch0, batch1, batch2, W, b):
    xs = [x0, x1, x2]
    eis = [edge_index0, edge_index1, edge_index2]
    bs = [batch0, batch1, batch2]
    ns = [x.shape[0] for x in xs]
    npads = [_round_up(n + 1, NTILES * CHUNK) for n in ns]
    epads = [_round_up(ei.shape[1], NTILES * CHUNK * NBUF) for ei in eis]

    wp = jnp.pad(W, ((0, 0), (0, LANES - W.shape[1])))
    zs = [_project(jnp.pad(x, ((0, npads[i] - ns[i]), (0, 0))), wp)
          for i, x in enumerate(xs)]
    es = []
    for i in range(3):
        alloc = epads[i] + NBUF * CHUNK
        pad = alloc - eis[i].shape[1]
        ep = jnp.pad(eis[i], ((0, 0), (0, pad)), constant_values=ns[i])
        es.append(ep.reshape(2, alloc // CHUNK, CHUNK).transpose(1, 0, 2))
    bpads = [jnp.pad(bs[i], (0, npads[i] - ns[i])) for i in range(3)]

    sc = _make_sc_kernel(npads, epads)
    pooled2 = sc(zs[0], zs[1], zs[2], es[0], es[1], es[2],
                 bpads[0], bpads[1], bpads[2])
    return pooled2.sum(axis=0)[:, : W.shape[1]] + b


# speedup vs baseline: 27.5676x; 1.1858x over previous
import functools

import jax
import jax.numpy as jnp
from jax import lax
from jax.experimental import pallas as pl
from jax.experimental.pallas import tpu as pltpu
from jax.experimental.pallas import tpu_sc as plsc

NUM_LAYERS = 3
LANES = 16
NTILES = 16
NCORES = 2
CHUNK = 128
NBUF = 8
BATCH = 64


def _round_up(a: int, m: int) -> int:
    return (a + m - 1) // m * m


def _mm_body(x_ref, w_ref, o_ref, *, n, bm):
    i = pl.program_id(0)
    rows = i * bm + lax.broadcasted_iota(jnp.int32, (bm, 1), 0)
    z = jnp.dot(x_ref[...], w_ref[...], preferred_element_type=jnp.float32)
    o_ref[...] = jnp.where(rows < n, z, 0.0)


def _project(x, wp, npad):
    n, d = x.shape
    bm = 2048
    return pl.pallas_call(
        functools.partial(_mm_body, n=n, bm=bm),
        grid=(npad // bm,),
        in_specs=[
            pl.BlockSpec((bm, d), lambda i: (i, 0)),
            pl.BlockSpec((d, LANES), lambda i: (0, 0)),
        ],
        out_specs=pl.BlockSpec((bm, LANES), lambda i: (i, 0)),
        out_shape=jax.ShapeDtypeStruct((npad, LANES), jnp.float32),
    )(x, wp)


def _make_sc_kernel(npads, epads):
    rpts = [n // NTILES for n in npads]
    ncts = [e // NTILES // CHUNK for e in epads]
    mesh = plsc.VectorSubcoreMesh(core_axis_name="c", subcore_axis_name="s")

    scr = dict(
        zA0=pltpu.VMEM_SHARED((npads[0], LANES), jnp.float32),
        zB0=pltpu.VMEM_SHARED((npads[0], LANES), jnp.float32),
        zA1=pltpu.VMEM_SHARED((npads[1], LANES), jnp.float32),
        zB1=pltpu.VMEM_SHARED((npads[1], LANES), jnp.float32),
        zA2=pltpu.VMEM_SHARED((npads[2], LANES), jnp.float32),
        zB2=pltpu.VMEM_SHARED((npads[2], LANES), jnp.float32),
        pooled=pltpu.VMEM_SHARED((BATCH, LANES), jnp.float32),
        eir0=pltpu.VMEM((2, NBUF, CHUNK), jnp.int32),
        eir1=pltpu.VMEM((2, NBUF, CHUNK), jnp.int32),
        pstage=pltpu.VMEM((BATCH, LANES), jnp.float32),
        isem0=pltpu.SemaphoreType.DMA,
        isem1=pltpu.SemaphoreType.DMA,
        isem0b=pltpu.SemaphoreType.DMA,
        isem1b=pltpu.SemaphoreType.DMA,
    )
    for bi in range(NBUF):
        scr[f"rbuf{bi}"] = pltpu.VMEM((CHUNK, LANES), jnp.float32)
    for bi in range(NBUF):
        scr[f"gsem{bi}"] = pltpu.SemaphoreType.DMA
    for bi in range(NBUF):
        scr[f"ssem{bi}"] = pltpu.SemaphoreType.DMA

    @functools.partial(
        pl.kernel,
        out_type=jax.ShapeDtypeStruct((NCORES, BATCH, LANES), jnp.float32),
        mesh=mesh,
        compiler_params=pltpu.CompilerParams(use_tc_tiling_on_sc=False),
        scratch_types=scr,
    )
    def sc_kernel(z0, z1, z2, e0, e1, e2, b0, b1, b2, out,
                  zA0, zB0, zA1, zB1, zA2, zB2, pooled,
                  eir0, eir1, pstage, isem0, isem1, isem0b, isem1b,
                  rbuf0, rbuf1, rbuf2, rbuf3, rbuf4, rbuf5, rbuf6, rbuf7,
                  gsem0, gsem1, gsem2, gsem3, gsem4, gsem5, gsem6, gsem7,
                  ssem0, ssem1, ssem2, ssem3, ssem4, ssem5, ssem6, ssem7):
        c = lax.axis_index("c")
        s = lax.axis_index("s")
        rbufs = [rbuf0, rbuf1, rbuf2, rbuf3, rbuf4, rbuf5, rbuf6, rbuf7]
        gsems = [gsem0, gsem1, gsem2, gsem3, gsem4, gsem5, gsem6, gsem7]
        ssems = [ssem0, ssem1, ssem2, ssem3, ssem4, ssem5, ssem6, ssem7]

        def grouped(n, issue_load, after_load):
            for g0 in range(0, n, NBUF):
                g = min(NBUF, n - g0)
                lds = [issue_load(g0 + i, i) for i in range(g)]
                sds = []
                for i in range(g):
                    lds[i].wait()
                    sds.extend(after_load(g0 + i, i))
                for sd in sds:
                    sd.wait()

        def load_dim(z_hbm, zA, zB, rpt):
            base = s * rpt

            def ld(j, i):
                sl = pl.ds(base + j * CHUNK, CHUNK)
                return pltpu.async_copy(z_hbm.at[sl], rbufs[i], gsems[i])

            def st(j, i):
                sl = pl.ds(base + j * CHUNK, CHUNK)
                return [pltpu.async_copy(rbufs[i], zA.at[sl], ssems[i]),
                        pltpu.async_copy(rbufs[i], zB.at[sl], gsems[i])]

            grouped(rpt // CHUNK, ld, st)

        def copy_rows(src, dst, rpt):
            base = s * rpt

            def ld(j, i):
                sl = pl.ds(base + j * CHUNK, CHUNK)
                return pltpu.async_copy(src.at[sl], rbufs[i], gsems[i])

            def st(j, i):
                sl = pl.ds(base + j * CHUNK, CHUNK)
                return [pltpu.async_copy(rbufs[i], dst.at[sl], ssems[i])]

            grouped(rpt // CHUNK, ld, st)

        def process_quad(zsrc, zdst, eir):
            gds = [
                pltpu.async_copy(zsrc.at[eir.at[0, bi]], rbufs[bi], gsems[bi])
                for bi in range(NBUF)
            ]
            sds = []
            for bi in range(NBUF):
                gds[bi].wait()
                sds.append(pltpu.async_copy(
                    rbufs[bi], zdst.at[eir.at[1, bi]], ssems[bi], add=True))
            for sd in sds:
                sd.wait()

        def edge_pass(zsrc, zdst, e_hbm, nct):
            base = s * nct
            nq = nct // NBUF
            pltpu.sync_copy(e_hbm.at[0, pl.ds(base, NBUF)], eir0.at[0])
            pltpu.sync_copy(e_hbm.at[1, pl.ds(base, NBUF)], eir0.at[1])

            def pair(h, carry):
                q1 = base + (2 * h + 1) * NBUF
                dB0 = pltpu.async_copy(e_hbm.at[0, pl.ds(q1, NBUF)],
                                       eir1.at[0], isem1)
                dB1 = pltpu.async_copy(e_hbm.at[1, pl.ds(q1, NBUF)],
                                       eir1.at[1], isem1b)
                process_quad(zsrc, zdst, eir0)
                dB0.wait()
                dB1.wait()
                dA0 = pltpu.async_copy(e_hbm.at[0, pl.ds(q1 + NBUF, NBUF)],
                                       eir0.at[0], isem0)
                dA1 = pltpu.async_copy(e_hbm.at[1, pl.ds(q1 + NBUF, NBUF)],
                                       eir0.at[1], isem0b)
                process_quad(zsrc, zdst, eir1)
                dA0.wait()
                dA1.wait()
                return carry

            lax.fori_loop(0, nq // 2, pair, 0)
            if nq % 2 == 1:
                process_quad(zsrc, zdst, eir0)

        def pool_dim(zfin, b_hbm, rpt):
            base = s * rpt

            def ld(j, i):
                sl = pl.ds(base + j * CHUNK, CHUNK)
                pltpu.sync_copy(b_hbm.at[sl], eir0.at[0, i])
                return pltpu.async_copy(zfin.at[sl], rbufs[i], gsems[i])

            def st(j, i):
                return [pltpu.async_copy(rbufs[i], pooled.at[eir0.at[0, i]],
                                         ssems[i], add=True)]

            grouped(rpt // CHUNK, ld, st)

        def run_dims(dims):
            for (zh, zA, zB, eh, bh, nct, rpt) in dims:
                load_dim(zh, zA, zB, rpt)

            @pl.when(s == 0)
            def _():
                zv = jnp.zeros((LANES,), jnp.float32)
                for i in range(BATCH):
                    pstage[i, :] = zv
                pltpu.sync_copy(pstage, pooled)

            plsc.subcore_barrier()

            for layer in range(NUM_LAYERS):
                fwd = layer % 2 == 0
                if layer > 0:
                    for (zh, zA, zB, eh, bh, nct, rpt) in dims:
                        copy_rows(zA if fwd else zB, zB if fwd else zA, rpt)
                    plsc.subcore_barrier()
                for (zh, zA, zB, eh, bh, nct, rpt) in dims:
                    edge_pass(zA if fwd else zB, zB if fwd else zA, eh, nct)
                plsc.subcore_barrier()

            for (zh, zA, zB, eh, bh, nct, rpt) in dims:
                pool_dim(zB if NUM_LAYERS % 2 == 1 else zA, bh, rpt)
            plsc.subcore_barrier()

            @pl.when(s == 0)
            def _():
                pltpu.sync_copy(pooled, pstage)
                pltpu.sync_copy(pstage, out.at[c])

        dim0 = (z0, zA0, zB0, e0, b0, ncts[0], rpts[0])
        dim1 = (z1, zA1, zB1, e1, b1, ncts[1], rpts[1])
        dim2 = (z2, zA2, zB2, e2, b2, ncts[2], rpts[2])

        @pl.when(c == 0)
        def _():
            run_dims([dim0])

        @pl.when(c == 1)
        def _():
            run_dims([dim1, dim2])

    return sc_kernel


def kernel(x0, x1, x2, edge_index0, edge_index1, edge_index2,
           batch0, batch1, batch2, W, b):
    xs = [x0, x1, x2]
    eis = [edge_index0, edge_index1, edge_index2]
    bs = [batch0, batch1, batch2]
    ns = [x.shape[0] for x in xs]
    npads = [_round_up(n + 1, NTILES * CHUNK) for n in ns]
    epads = [_round_up(ei.shape[1], NTILES * CHUNK * NBUF) for ei in eis]

    wp = jnp.pad(W, ((0, 0), (0, LANES - W.shape[1])))
    zs = [_project(xs[i], wp, npads[i]) for i in range(3)]
    es = []
    for i in range(3):
        alloc = epads[i] + NBUF * CHUNK
        pad = alloc - eis[i].shape[1]
        ep = jnp.pad(eis[i], ((0, 0), (0, pad)), constant_values=ns[i])
        es.append(ep.reshape(2, alloc // CHUNK, CHUNK))
    bpads = [jnp.pad(bs[i], (0, npads[i] - ns[i])) for i in range(3)]

    sc = _make_sc_kernel(npads, epads)
    pooled2 = sc(zs[0], zs[1], zs[2], es[0], es[1], es[2],
                 bpads[0], bpads[1], bpads[2])
    return pooled2.sum(axis=0)[:, : W.shape[1]] + b


# speedup vs baseline: 28.5701x; 1.0364x over previous
import functools

import jax
import jax.numpy as jnp
from jax import lax
from jax.experimental import pallas as pl
from jax.experimental.pallas import tpu as pltpu
from jax.experimental.pallas import tpu_sc as plsc

NUM_LAYERS = 3
LANES = 16
NTILES = 16
NCORES = 2
CHUNK = 128
NBUF = 8
BATCH = 64


def _round_up(a: int, m: int) -> int:
    return (a + m - 1) // m * m


def _mm_body(x_ref, w_ref, o_ref, *, n, bm):
    i = pl.program_id(0)
    rows = i * bm + lax.broadcasted_iota(jnp.int32, (bm, 1), 0)
    z = jnp.dot(x_ref[...], w_ref[...], preferred_element_type=jnp.float32)
    o_ref[...] = jnp.where(rows < n, z, 0.0)


def _project(x, wp, npad):
    n, d = x.shape
    bm = npad // 4 if npad % 4096 == 0 else npad // 2
    while npad // bm * bm != npad:
        bm //= 2
    return pl.pallas_call(
        functools.partial(_mm_body, n=n, bm=bm),
        grid=(npad // bm,),
        in_specs=[
            pl.BlockSpec((bm, d), lambda i: (i, 0)),
            pl.BlockSpec((d, LANES), lambda i: (0, 0)),
        ],
        out_specs=pl.BlockSpec((bm, LANES), lambda i: (i, 0)),
        out_shape=jax.ShapeDtypeStruct((npad, LANES), jnp.float32),
    )(x, wp)


def _make_sc_kernel(npads, epads):
    rpts = [n // NTILES for n in npads]
    ncts = [e // NTILES // CHUNK for e in epads]
    mesh = plsc.VectorSubcoreMesh(core_axis_name="c", subcore_axis_name="s")

    scr = dict(
        zA0=pltpu.VMEM_SHARED((npads[0], LANES), jnp.float32),
        zB0=pltpu.VMEM_SHARED((npads[0], LANES), jnp.float32),
        zA1=pltpu.VMEM_SHARED((npads[1], LANES), jnp.float32),
        zB1=pltpu.VMEM_SHARED((npads[1], LANES), jnp.float32),
        zA2=pltpu.VMEM_SHARED((npads[2], LANES), jnp.float32),
        zB2=pltpu.VMEM_SHARED((npads[2], LANES), jnp.float32),
        pooled=pltpu.VMEM_SHARED((BATCH, LANES), jnp.float32),
        eir0=pltpu.VMEM((2, NBUF, CHUNK), jnp.int32),
        eir1=pltpu.VMEM((2, NBUF, CHUNK), jnp.int32),
        pstage=pltpu.VMEM((BATCH, LANES), jnp.float32),
        isem0=pltpu.SemaphoreType.DMA,
        isem1=pltpu.SemaphoreType.DMA,
        isem0b=pltpu.SemaphoreType.DMA,
        isem1b=pltpu.SemaphoreType.DMA,
    )
    for bi in range(NBUF):
        scr[f"rbuf{bi}"] = pltpu.VMEM((CHUNK, LANES), jnp.float32)
    for bi in range(NBUF):
        scr[f"gsem{bi}"] = pltpu.SemaphoreType.DMA
    for bi in range(NBUF):
        scr[f"ssem{bi}"] = pltpu.SemaphoreType.DMA

    @functools.partial(
        pl.kernel,
        out_type=jax.ShapeDtypeStruct((NCORES, BATCH, LANES), jnp.float32),
        mesh=mesh,
        compiler_params=pltpu.CompilerParams(use_tc_tiling_on_sc=False),
        scratch_types=scr,
    )
    def sc_kernel(z0, z1, z2, e0, e1, e2, b0, b1, b2, out,
                  zA0, zB0, zA1, zB1, zA2, zB2, pooled,
                  eir0, eir1, pstage, isem0, isem1, isem0b, isem1b,
                  rbuf0, rbuf1, rbuf2, rbuf3, rbuf4, rbuf5, rbuf6, rbuf7,
                  gsem0, gsem1, gsem2, gsem3, gsem4, gsem5, gsem6, gsem7,
                  ssem0, ssem1, ssem2, ssem3, ssem4, ssem5, ssem6, ssem7):
        c = lax.axis_index("c")
        s = lax.axis_index("s")
        rbufs = [rbuf0, rbuf1, rbuf2, rbuf3, rbuf4, rbuf5, rbuf6, rbuf7]
        gsems = [gsem0, gsem1, gsem2, gsem3, gsem4, gsem5, gsem6, gsem7]
        ssems = [ssem0, ssem1, ssem2, ssem3, ssem4, ssem5, ssem6, ssem7]

        def grouped(n, issue_load, after_load):
            for g0 in range(0, n, NBUF):
                g = min(NBUF, n - g0)
                lds = [issue_load(g0 + i, i) for i in range(g)]
                sds = []
                for i in range(g):
                    lds[i].wait()
                    sds.extend(after_load(g0 + i, i))
                for sd in sds:
                    sd.wait()

        def load_dim(z_hbm, zA, zB, rpt):
            base = s * rpt

            def ld(j, i):
                sl = pl.ds(base + j * CHUNK, CHUNK)
                return pltpu.async_copy(z_hbm.at[sl], rbufs[i], gsems[i])

            def st(j, i):
                sl = pl.ds(base + j * CHUNK, CHUNK)
                return [pltpu.async_copy(rbufs[i], zA.at[sl], ssems[i]),
                        pltpu.async_copy(rbufs[i], zB.at[sl], gsems[i])]

            grouped(rpt // CHUNK, ld, st)

        def copy_rows(src, dst, rpt):
            base = s * rpt

            def ld(j, i):
                sl = pl.ds(base + j * CHUNK, CHUNK)
                return pltpu.async_copy(src.at[sl], rbufs[i], gsems[i])

            def st(j, i):
                sl = pl.ds(base + j * CHUNK, CHUNK)
                return [pltpu.async_copy(rbufs[i], dst.at[sl], ssems[i])]

            grouped(rpt // CHUNK, ld, st)

        def process_quad(zsrc, zdst, eir):
            gds = [
                pltpu.async_copy(zsrc.at[eir.at[0, bi]], rbufs[bi], gsems[bi])
                for bi in range(NBUF)
            ]
            sds = []
            for bi in range(NBUF):
                gds[bi].wait()
                sds.append(pltpu.async_copy(
                    rbufs[bi], zdst.at[eir.at[1, bi]], ssems[bi], add=True))
            for sd in sds:
                sd.wait()

        def edge_pass(zsrc, zdst, e_hbm, nct):
            base = s * nct
            nq = nct // NBUF
            pltpu.sync_copy(e_hbm.at[0, pl.ds(base, NBUF)], eir0.at[0])
            pltpu.sync_copy(e_hbm.at[1, pl.ds(base, NBUF)], eir0.at[1])

            def pair(h, carry):
                q1 = base + (2 * h + 1) * NBUF
                dB0 = pltpu.async_copy(e_hbm.at[0, pl.ds(q1, NBUF)],
                                       eir1.at[0], isem1)
                dB1 = pltpu.async_copy(e_hbm.at[1, pl.ds(q1, NBUF)],
                                       eir1.at[1], isem1b)
                process_quad(zsrc, zdst, eir0)
                dB0.wait()
                dB1.wait()
                dA0 = pltpu.async_copy(e_hbm.at[0, pl.ds(q1 + NBUF, NBUF)],
                                       eir0.at[0], isem0)
                dA1 = pltpu.async_copy(e_hbm.at[1, pl.ds(q1 + NBUF, NBUF)],
                                       eir0.at[1], isem0b)
                process_quad(zsrc, zdst, eir1)
                dA0.wait()
                dA1.wait()
                return carry

            lax.fori_loop(0, nq // 2, pair, 0)
            if nq % 2 == 1:
                process_quad(zsrc, zdst, eir0)

        def pool_dim(zfin, b_hbm, rpt):
            base = s * rpt

            def ld(j, i):
                sl = pl.ds(base + j * CHUNK, CHUNK)
                pltpu.sync_copy(b_hbm.at[sl], eir0.at[0, i])
                return pltpu.async_copy(zfin.at[sl], rbufs[i], gsems[i])

            def st(j, i):
                return [pltpu.async_copy(rbufs[i], pooled.at[eir0.at[0, i]],
                                         ssems[i], add=True)]

            grouped(rpt // CHUNK, ld, st)

        def run_dims(dims):
            for (zh, zA, zB, eh, bh, nct, rpt) in dims:
                load_dim(zh, zA, zB, rpt)

            @pl.when(s == 0)
            def _():
                zv = jnp.zeros((LANES,), jnp.float32)
                for i in range(BATCH):
                    pstage[i, :] = zv
                pltpu.sync_copy(pstage, pooled)

            plsc.subcore_barrier()

            for layer in range(NUM_LAYERS):
                fwd = layer % 2 == 0
                if layer > 0:
                    for (zh, zA, zB, eh, bh, nct, rpt) in dims:
                        copy_rows(zA if fwd else zB, zB if fwd else zA, rpt)
                    plsc.subcore_barrier()
                for (zh, zA, zB, eh, bh, nct, rpt) in dims:
                    edge_pass(zA if fwd else zB, zB if fwd else zA, eh, nct)
                plsc.subcore_barrier()

            for (zh, zA, zB, eh, bh, nct, rpt) in dims:
                pool_dim(zB if NUM_LAYERS % 2 == 1 else zA, bh, rpt)
            plsc.subcore_barrier()

            @pl.when(s == 0)
            def _():
                pltpu.sync_copy(pooled, pstage)
                pltpu.sync_copy(pstage, out.at[c])

        dim0 = (z0, zA0, zB0, e0, b0, ncts[0], rpts[0])
        dim1 = (z1, zA1, zB1, e1, b1, ncts[1], rpts[1])
        dim2 = (z2, zA2, zB2, e2, b2, ncts[2], rpts[2])

        @pl.when(c == 0)
        def _():
            run_dims([dim0])

        @pl.when(c == 1)
        def _():
            run_dims([dim1, dim2])

    return sc_kernel


def kernel(x0, x1, x2, edge_index0, edge_index1, edge_index2,
           batch0, batch1, batch2, W, b):
    xs = [x0, x1, x2]
    eis = [edge_index0, edge_index1, edge_index2]
    bs = [batch0, batch1, batch2]
    ns = [x.shape[0] for x in xs]
    npads = [_round_up(n + 1, NTILES * CHUNK) for n in ns]
    epads = [_round_up(ei.shape[1], NTILES * CHUNK * NBUF) for ei in eis]

    wp = jnp.pad(W, ((0, 0), (0, LANES - W.shape[1])))
    zs = [_project(xs[i], wp, npads[i]) for i in range(3)]
    es = []
    for i in range(3):
        alloc = epads[i] + NBUF * CHUNK
        pad = alloc - eis[i].shape[1]
        ep = jnp.pad(eis[i], ((0, 0), (0, pad)), constant_values=ns[i])
        es.append(ep.reshape(2, alloc // CHUNK, CHUNK))
    bpads = [jnp.pad(bs[i], (0, npads[i] - ns[i])) for i in range(3)]

    sc = _make_sc_kernel(npads, epads)
    pooled2 = sc(zs[0], zs[1], zs[2], es[0], es[1], es[2],
                 bpads[0], bpads[1], bpads[2])
    return pooled2.sum(axis=0)[:, : W.shape[1]] + b
